# Initial kernel scaffold; baseline (speedup 1.0000x reference)
import jax
import jax.numpy as jnp
from jax.experimental import pallas as pl


def kernel(cls_pred, cls_target):
    raise NotImplementedError("write your pallas kernel here")


# speedup vs baseline: 1.4925x; 1.4925x over previous
import functools

import jax
import jax.numpy as jnp
from jax.experimental import pallas as pl

RATE = 0.7
ROWS_PER_BLOCK = 512


def _ce_block_kernel(x_ref, tgt_ref, ce_ref):
    x = x_ref[...]
    tgt = tgt_ref[...]
    m = jnp.max(x, axis=1, keepdims=True)
    s = jnp.sum(jnp.exp(x - m), axis=1, keepdims=True)
    lse = m + jnp.log(s)
    col = jax.lax.broadcasted_iota(jnp.int32, x.shape, 1)
    tv = jnp.sum(jnp.where(col == tgt, x, 0.0), axis=1, keepdims=True)
    ce = lse - tv
    ce = jnp.where(tgt == -1, 0.0, ce)
    ce_ref[...] = jnp.maximum(ce, 0.0)


def _topk_sum_kernel(ce_ref, out_ref, *, keep_num):
    ce = ce_ref[...]
    v = jax.lax.bitcast_convert_type(ce, jnp.int32)

    def body(j, t):
        b = 30 - j
        cand = t | (jnp.int32(1) << b)
        cnt = jnp.sum((v >= cand).astype(jnp.int32))
        return jnp.where(cnt >= keep_num, cand, t)

    t = jax.lax.fori_loop(0, 31, body, jnp.int32(0))
    t_f = jax.lax.bitcast_convert_type(t, jnp.float32)
    cnt_gt = jnp.sum((v > t).astype(jnp.int32))
    sum_gt = jnp.sum(jnp.where(v > t, ce, 0.0))
    total = sum_gt + (keep_num - cnt_gt).astype(jnp.float32) * t_f
    out_ref[...] = jnp.broadcast_to(total / keep_num, (1, 1))


def kernel(cls_pred, cls_target):
    n, c = cls_pred.shape
    keep_num = min(n, int(n * RATE))
    tgt = cls_target.astype(jnp.int32).reshape(n, 1)

    r = ROWS_PER_BLOCK
    nb = n // r
    ce = pl.pallas_call(
        _ce_block_kernel,
        grid=(nb,),
        in_specs=[
            pl.BlockSpec((r, c), lambda i: (i, 0)),
            pl.BlockSpec((r, 1), lambda i: (i, 0)),
        ],
        out_specs=pl.BlockSpec((r, 1), lambda i: (i, 0)),
        out_shape=jax.ShapeDtypeStruct((n, 1), jnp.float32),
    )(cls_pred, tgt)

    ce2 = ce.reshape(128, n // 128)
    loss = pl.pallas_call(
        functools.partial(_topk_sum_kernel, keep_num=keep_num),
        out_shape=jax.ShapeDtypeStruct((1, 1), jnp.float32),
    )(ce2)
    return loss.reshape(())
